# baseline (speedup 1.0000x reference)
import jax
import jax.numpy as jnp
from jax import lax
from jax.experimental import pallas as pl
from jax.experimental.pallas import tpu as pltpu
from jax.experimental.pallas import tpu_sc as plsc

S = 197
B = S * S
HALF = 384
CHUNK = 128
NFULL = B // CHUNK
NCHUNK = NFULL + 1
TBASE = B - CHUNK

_info = plsc.get_sparse_core_info()
_NC, _NS = _info.num_cores, _info.num_subcores
NW = _NC * _NS
ITERS = -(-NCHUNK // NW)


def _body(xt_hbm, yt_hbm, xi_hbm, yi_hbm, xit_hbm, yit_hbm, out_hbm,
          xi_v, yi_v, xrow_v, yrow_v, sem):
    wid = lax.axis_index("s") * _NC + lax.axis_index("c")

    def step(t, carry):
        c = wid + NW * t

        @pl.when(c < NFULL)
        def _():
            pltpu.sync_copy(xi_hbm.at[pl.ds(c * CHUNK, CHUNK)], xi_v)
            pltpu.sync_copy(yi_hbm.at[pl.ds(c * CHUNK, CHUNK)], yi_v)

        @pl.when(c == NFULL)
        def _():
            pltpu.sync_copy(xit_hbm, xi_v)
            pltpu.sync_copy(yit_hbm, yi_v)

        @pl.when(c < NCHUNK)
        def _():
            base = jnp.where(c == NFULL, TBASE, c * CHUNK)
            cpx = pltpu.async_copy(xt_hbm.at[xi_v], xrow_v, sem)
            cpy = pltpu.async_copy(yt_hbm.at[yi_v], yrow_v, sem)
            cpx.wait()
            cpy.wait()
            pltpu.sync_copy(xrow_v, out_hbm.at[pl.ds(base, CHUNK), 0])
            pltpu.sync_copy(yrow_v, out_hbm.at[pl.ds(base, CHUNK), 1])

        return carry

    lax.fori_loop(0, ITERS, step, 0)


def kernel(x_table, y_table, x_dis, y_dis):
    xi = x_dis.reshape(-1)
    yi = y_dis.reshape(-1)
    run = pl.kernel(
        _body,
        out_type=jax.ShapeDtypeStruct((B, 2, HALF), jnp.float32),
        mesh=plsc.VectorSubcoreMesh(core_axis_name="c", subcore_axis_name="s"),
        scratch_types=[
            pltpu.VMEM((CHUNK,), jnp.int32),
            pltpu.VMEM((CHUNK,), jnp.int32),
            pltpu.VMEM((CHUNK, HALF), jnp.float32),
            pltpu.VMEM((CHUNK, HALF), jnp.float32),
            pltpu.SemaphoreType.DMA,
        ],
    )
    out = run(x_table, y_table, xi, yi, xi[TBASE:], yi[TBASE:])
    return out.reshape(S, S, 2 * HALF)


# speedup vs baseline: 1.0540x; 1.0540x over previous
import jax
import jax.numpy as jnp
from jax import lax
from jax.experimental import pallas as pl
from jax.experimental.pallas import tpu as pltpu
from jax.experimental.pallas import tpu_sc as plsc

S = 197
B = S * S
HALF = 384
CHUNK = 64
NFULL = B // CHUNK
NCHUNK = NFULL + 1
TBASE = B - CHUNK

_info = plsc.get_sparse_core_info()
_NC, _NS = _info.num_cores, _info.num_subcores
NW = _NC * _NS
K = -(-NCHUNK // NW)
PAD_B = NW * K * CHUNK


def _body(xt_hbm, yt_hbm, xi_hbm, yi_hbm, xit_hbm, yit_hbm, out_hbm,
          xi_v, yi_v, xb0, xb1, yb0, yb1,
          gx0, gx1, gy0, gy1, w0, w1):
    wid = lax.axis_index("s") * _NC + lax.axis_index("c")
    last = wid == NW - 1

    xbuf, ybuf = (xb0, xb1), (yb0, yb1)
    gx, gy, wsem = (gx0, gx1), (gy0, gy1), (w0, w1)

    pltpu.sync_copy(xi_hbm.at[wid], xi_v)
    pltpu.sync_copy(yi_hbm.at[wid], yi_v)

    @pl.when(last)
    def _():
        pltpu.sync_copy(xit_hbm, xi_v.at[K - 2])
        pltpu.sync_copy(yit_hbm, yi_v.at[K - 2])

    def make_g(k):
        b = k % 2
        return (pltpu.make_async_copy(xt_hbm.at[xi_v.at[k]], xbuf[b], gx[b]),
                pltpu.make_async_copy(yt_hbm.at[yi_v.at[k]], ybuf[b], gy[b]))

    def make_w(k):
        b = k % 2
        c = wid * K + k
        base = jnp.where(c == NFULL, TBASE, c * CHUNK)
        return (pltpu.make_async_copy(xbuf[b],
                                      out_hbm.at[pl.ds(base, CHUNK), 0],
                                      wsem[b]),
                pltpu.make_async_copy(ybuf[b],
                                      out_hbm.at[pl.ds(base, CHUNK), 1],
                                      wsem[b]))

    g = [make_g(k) for k in range(K)]
    w = [make_w(k) for k in range(K)]

    def start(ops):
        ops[0].start()
        ops[1].start()

    def wait(ops):
        ops[0].wait()
        ops[1].wait()

    for k in range(K):
        guard = (k == K - 1)
        if k >= 2:
            wait(w[k - 2])
        if guard:
            @pl.when(~last)
            def _():
                start(g[k])
        else:
            start(g[k])
        if k >= 1:
            wait(g[k - 1])
            start(w[k - 1])

    @pl.when(~last)
    def _():
        wait(g[K - 1])
        start(w[K - 1])

    wait(w[K - 2])

    @pl.when(~last)
    def _():
        wait(w[K - 1])


def kernel(x_table, y_table, x_dis, y_dis):
    xi = x_dis.reshape(-1)
    yi = y_dis.reshape(-1)
    pad = PAD_B - B
    xi_r = jnp.pad(xi, (0, pad)).reshape(NW, K, CHUNK)
    yi_r = jnp.pad(yi, (0, pad)).reshape(NW, K, CHUNK)
    run = pl.kernel(
        _body,
        out_type=jax.ShapeDtypeStruct((B, 2, HALF), jnp.float32),
        mesh=plsc.VectorSubcoreMesh(core_axis_name="c", subcore_axis_name="s"),
        scratch_types=[
            pltpu.VMEM((K, CHUNK), jnp.int32),
            pltpu.VMEM((K, CHUNK), jnp.int32),
            pltpu.VMEM((CHUNK, HALF), jnp.float32),
            pltpu.VMEM((CHUNK, HALF), jnp.float32),
            pltpu.VMEM((CHUNK, HALF), jnp.float32),
            pltpu.VMEM((CHUNK, HALF), jnp.float32),
            pltpu.SemaphoreType.DMA,
            pltpu.SemaphoreType.DMA,
            pltpu.SemaphoreType.DMA,
            pltpu.SemaphoreType.DMA,
            pltpu.SemaphoreType.DMA,
            pltpu.SemaphoreType.DMA,
        ],
    )
    out = run(x_table, y_table, xi_r, yi_r, xi[TBASE:], yi[TBASE:])
    return out.reshape(S, S, 2 * HALF)


# speedup vs baseline: 1.4487x; 1.3745x over previous
import jax
import jax.numpy as jnp
from jax import lax
from jax.experimental import pallas as pl
from jax.experimental.pallas import tpu as pltpu
from jax.experimental.pallas import tpu_sc as plsc

S = 197
B = S * S
HALF = 384
CHUNK = 64
NFULL = B // CHUNK
NCHUNK = NFULL + 1
TBASE = B - CHUNK

_info = plsc.get_sparse_core_info()
_NC, _NS = _info.num_cores, _info.num_subcores
NW = _NC * _NS
K = -(-NCHUNK // NW)
PAD_B = NW * K * CHUNK


def _body(xt_hbm, yt_hbm, xi_hbm, yi_hbm, xit_hbm, yit_hbm, out_hbm,
          xi_v, yi_v, xb0, xb1, yb0, yb1,
          gx0, gx1, gy0, gy1, w0, w1):
    wid = lax.axis_index("s") * _NC + lax.axis_index("c")
    last = wid == NW - 1

    xbuf, ybuf = (xb0, xb1), (yb0, yb1)
    gx, gy, wsem = (gx0, gx1), (gy0, gy1), (w0, w1)

    pltpu.sync_copy(xi_hbm.at[wid], xi_v)
    pltpu.sync_copy(yi_hbm.at[wid], yi_v)

    @pl.when(last)
    def _():
        pltpu.sync_copy(xit_hbm, xi_v.at[K - 2])
        pltpu.sync_copy(yit_hbm, yi_v.at[K - 2])

    def make_g(k):
        b = k % 2
        return (pltpu.make_async_copy(xt_hbm.at[xi_v.at[k]], xbuf[b], gx[b]),
                pltpu.make_async_copy(yt_hbm.at[yi_v.at[k]], ybuf[b], gy[b]))

    def make_w(k):
        b = k % 2
        c = wid * K + k
        base = jnp.where(c == NFULL, TBASE, c * CHUNK)
        return (pltpu.make_async_copy(xbuf[b],
                                      out_hbm.at[pl.ds(base, CHUNK), 0],
                                      wsem[b]),
                pltpu.make_async_copy(ybuf[b],
                                      out_hbm.at[pl.ds(base, CHUNK), 1],
                                      wsem[b]))

    g = [make_g(k) for k in range(K)]
    w = [make_w(k) for k in range(K)]

    def start(ops):
        ops[0].start()
        ops[1].start()

    def wait(ops):
        ops[0].wait()
        ops[1].wait()

    for k in range(K):
        guard = (k == K - 1)
        if k >= 2:
            wait(w[k - 2])
        if guard:
            @pl.when(~last)
            def _():
                start(g[k])
        else:
            start(g[k])
        if k >= 1:
            wait(g[k - 1])
            start(w[k - 1])

    @pl.when(~last)
    def _():
        wait(g[K - 1])
        start(w[K - 1])

    wait(w[K - 2])

    @pl.when(~last)
    def _():
        wait(w[K - 1])


def kernel(x_table, y_table, x_dis, y_dis):
    rows = x_table.shape[0]
    xt_rep = jnp.tile(x_table, (NW, 1))
    yt_rep = jnp.tile(y_table, (NW, 1))
    xi = x_dis.reshape(-1)
    yi = y_dis.reshape(-1)
    pad = PAD_B - B
    woff = (jnp.arange(PAD_B, dtype=jnp.int32) // (K * CHUNK)) * rows
    xi_r = (jnp.pad(xi, (0, pad)) + woff).reshape(NW, K, CHUNK)
    yi_r = (jnp.pad(yi, (0, pad)) + woff).reshape(NW, K, CHUNK)
    toff = (NW - 1) * rows
    run = pl.kernel(
        _body,
        out_type=jax.ShapeDtypeStruct((B, 2, HALF), jnp.float32),
        mesh=plsc.VectorSubcoreMesh(core_axis_name="c", subcore_axis_name="s"),
        scratch_types=[
            pltpu.VMEM((K, CHUNK), jnp.int32),
            pltpu.VMEM((K, CHUNK), jnp.int32),
            pltpu.VMEM((CHUNK, HALF), jnp.float32),
            pltpu.VMEM((CHUNK, HALF), jnp.float32),
            pltpu.VMEM((CHUNK, HALF), jnp.float32),
            pltpu.VMEM((CHUNK, HALF), jnp.float32),
            pltpu.SemaphoreType.DMA,
            pltpu.SemaphoreType.DMA,
            pltpu.SemaphoreType.DMA,
            pltpu.SemaphoreType.DMA,
            pltpu.SemaphoreType.DMA,
            pltpu.SemaphoreType.DMA,
        ],
    )
    out = run(xt_rep, yt_rep, xi_r, yi_r, xi[TBASE:] + toff, yi[TBASE:] + toff)
    return out.reshape(S, S, 2 * HALF)
